# baseline (speedup 1.0000x reference)
import functools
import jax
import jax.numpy as jnp
from jax import lax
from jax.experimental import pallas as pl
from jax.experimental.pallas import tpu as pltpu
from jax.experimental.pallas import tpu_sc as plsc

_N, _E, _D = 4096, 8, 768
_BB = 512
_NC, _NS, _L = 2, 16, 16
_NW = _NC * _NS
_VPW = _N * _E // _NW


def _gate_of(xv):
    t = xv * 2.0
    t2 = t * t
    return 1.0 / (1.0 + t2 * t2)


def _sc_gate_body(x_hbm, gate_hbm, x_v, g_v):
    wid = lax.axis_index("s") * _NC + lax.axis_index("c")
    base = wid * _VPW
    pltpu.sync_copy(x_hbm.at[pl.ds(base, _VPW)], x_v)
    for i in range(_VPW // _L):
        g_v[pl.ds(i * _L, _L)] = _gate_of(x_v[pl.ds(i * _L, _L)])
    pltpu.sync_copy(g_v, gate_hbm.at[pl.ds(base, _VPW)])


def _sc_gate(x_flat):
    run = pl.kernel(
        _sc_gate_body,
        out_type=jax.ShapeDtypeStruct((_N * _E,), jnp.float32),
        mesh=plsc.VectorSubcoreMesh(core_axis_name="c", subcore_axis_name="s"),
        scratch_types=[
            pltpu.VMEM((_VPW,), jnp.float32),
            pltpu.VMEM((_VPW,), jnp.float32),
        ],
    )
    return run(x_flat)


def _tc_body(x_ref, act_ref, disp_ref):
    gate = _gate_of(x_ref[...])
    gm = jnp.where(gate >= 0.5, gate, 0.0)
    a = act_ref[...]
    disp_ref[...] = gm[:, :, None] * a[:, None, :]


def _tc_disp(x, act):
    return pl.pallas_call(
        _tc_body,
        grid=(_N // _BB,),
        in_specs=[
            pl.BlockSpec((_BB, _E), lambda i: (i, 0)),
            pl.BlockSpec((_BB, _D), lambda i: (i, 0)),
        ],
        out_specs=pl.BlockSpec((_BB, _E, _D), lambda i: (i, 0, 0)),
        out_shape=jax.ShapeDtypeStruct((_N, _E, _D), jnp.float32),
    )(x, act)


def kernel(x, act, batch_inds):
    gate = _sc_gate(x.reshape(_N * _E)).reshape(_N, _E)
    disp = _tc_disp(x, act)
    return gate, disp


# speedup vs baseline: 1.0009x; 1.0009x over previous
import functools
import jax
import jax.numpy as jnp
from jax import lax
from jax.experimental import pallas as pl
from jax.experimental.pallas import tpu as pltpu
from jax.experimental.pallas import tpu_sc as plsc

_N, _E, _D = 4096, 8, 768
_BB = 512
_NC, _NS, _L = 2, 16, 16
_NW = _NC * _NS
_VPW = _N * _E // _NW


def _gate_of(xv):
    t = xv * 2.0
    t2 = t * t
    return 1.0 / (1.0 + t2 * t2)


def _sc_gate_body(x_hbm, gate_hbm, x_v, g_v):
    wid = lax.axis_index("s") * _NC + lax.axis_index("c")
    base = wid * _VPW
    pltpu.sync_copy(x_hbm.at[pl.ds(base, _VPW)], x_v)
    for i in range(_VPW // _L):
        g_v[pl.ds(i * _L, _L)] = _gate_of(x_v[pl.ds(i * _L, _L)])
    pltpu.sync_copy(g_v, gate_hbm.at[pl.ds(base, _VPW)])


def _sc_gate(x_flat):
    run = pl.kernel(
        _sc_gate_body,
        out_type=jax.ShapeDtypeStruct((_N * _E,), jnp.float32),
        mesh=plsc.VectorSubcoreMesh(core_axis_name="c", subcore_axis_name="s"),
        scratch_types=[
            pltpu.VMEM((_VPW,), jnp.float32),
            pltpu.VMEM((_VPW,), jnp.float32),
        ],
    )
    return run(x_flat)


def _tc_body(x_ref, act_ref, disp_ref):
    gate = _gate_of(x_ref[...])
    gm = jnp.where(gate >= 0.5, gate, 0.0)
    a = act_ref[...]
    disp_ref[...] = gm[:, :, None] * a[:, None, :]


def _tc_disp(x, act):
    return pl.pallas_call(
        _tc_body,
        grid=(_N // _BB,),
        in_specs=[
            pl.BlockSpec((_BB, _E), lambda i: (i, 0)),
            pl.BlockSpec((_BB, _D), lambda i: (i, 0)),
        ],
        out_specs=pl.BlockSpec((_BB, _E, _D), lambda i: (i, 0, 0)),
        out_shape=jax.ShapeDtypeStruct((_N, _E, _D), jnp.float32),
    )(x, act)


def kernel(x, act, batch_inds):
    disp = _tc_disp(x, act)
    gate = _sc_gate(x.reshape(_N * _E)).reshape(_N, _E)
    return gate, disp


# speedup vs baseline: 1.3651x; 1.3639x over previous
import jax
import jax.numpy as jnp
from jax import lax
from jax.experimental import pallas as pl
from jax.experimental.pallas import tpu as pltpu

_N, _E, _D = 4096, 8, 768
_CB = 512
_NCH = _N // _CB
_NBUF = 4
_LOOK = 2


def _body(x_hbm, act_hbm, gate_hbm, disp_hbm,
          x_v, gate_v, act_b, disp_b, in_sems, out_sems, out_sems2, gsem):
    pltpu.make_async_copy(x_hbm, x_v, gsem).start()
    pltpu.make_async_copy(x_hbm, x_v, gsem).wait()
    t = x_v[...] * 2.0
    t2 = t * t
    gate_v[...] = 1.0 / (1.0 + t2 * t2)
    pltpu.make_async_copy(gate_v, gate_hbm, gsem).start()

    def act_in(c, slot):
        return pltpu.make_async_copy(
            act_hbm.at[pl.ds(c * _CB, _CB)], act_b.at[slot], in_sems.at[slot])

    _H = _CB // 2

    def disp_out_h(c, slot, h, sems):
        return pltpu.make_async_copy(
            disp_b.at[slot, pl.ds(h * _H, _H)],
            disp_hbm.at[pl.ds(c * _CB + h * _H, _H)], sems.at[slot])

    for c in range(_LOOK):
        act_in(c, c % _NBUF).start()

    def step(c, carry):
        slot = lax.rem(c, _NBUF)

        @pl.when(c + _LOOK < _NCH)
        def _():
            act_in(c + _LOOK, lax.rem(c + _LOOK, _NBUF)).start()

        act_in(c, slot).wait()

        @pl.when(c >= _NBUF)
        def _():
            disp_out_h(c - _NBUF, slot, 0, out_sems).wait()
            disp_out_h(c - _NBUF, slot, 1, out_sems2).wait()

        for h, sems in ((0, out_sems), (1, out_sems2)):
            gate = gate_v[pl.ds(c * _CB + h * _H, _H), :]
            gm = jnp.where(gate >= 0.5, gate, 0.0)
            a = act_b[slot, pl.ds(h * _H, _H)]
            disp_b[slot, pl.ds(h * _H, _H)] = gm[:, :, None] * a[:, None, :]
            disp_out_h(c, slot, h, sems).start()
        return carry

    lax.fori_loop(0, _NCH, step, 0, unroll=False)

    for k in range(_NCH - _NBUF, _NCH):
        disp_out_h(k, k % _NBUF, 0, out_sems).wait()
        disp_out_h(k, k % _NBUF, 1, out_sems2).wait()
    pltpu.make_async_copy(gate_v, gate_hbm, gsem).wait()


def kernel(x, act, batch_inds):
    gate, disp = pl.pallas_call(
        _body,
        in_specs=[
            pl.BlockSpec(memory_space=pl.ANY),
            pl.BlockSpec(memory_space=pl.ANY),
        ],
        out_specs=[
            pl.BlockSpec(memory_space=pl.ANY),
            pl.BlockSpec(memory_space=pl.ANY),
        ],
        out_shape=[
            jax.ShapeDtypeStruct((_N, _E), jnp.float32),
            jax.ShapeDtypeStruct((_N, _E, _D), jnp.float32),
        ],
        scratch_shapes=[
            pltpu.VMEM((_N, _E), jnp.float32),
            pltpu.VMEM((_N, _E), jnp.float32),
            pltpu.VMEM((_NBUF, _CB, _D), jnp.float32),
            pltpu.VMEM((_NBUF, _CB, _E, _D), jnp.float32),
            pltpu.SemaphoreType.DMA((_NBUF,)),
            pltpu.SemaphoreType.DMA((_NBUF,)),
            pltpu.SemaphoreType.DMA((_NBUF,)),
            pltpu.SemaphoreType.DMA,
        ],
    )(x, act)
    return gate, disp


# speedup vs baseline: 1.3670x; 1.0014x over previous
import jax
import jax.numpy as jnp
from jax import lax
from jax.experimental import pallas as pl
from jax.experimental.pallas import tpu as pltpu

_N, _E, _D = 4096, 8, 768
_CB = 512
_NCH = _N // _CB
_NBUF = 4
_LOOK = 2


def _body(x_hbm, act_hbm, gate_hbm, disp_hbm,
          x_v, gate_v, act_b, disp_b, in_sems, out_sems, out_sems2, gsem):
    pltpu.make_async_copy(x_hbm, x_v, gsem).start()
    pltpu.make_async_copy(x_hbm, x_v, gsem).wait()
    t = x_v[...] * 2.0
    t2 = t * t
    gate_v[...] = 1.0 / (1.0 + t2 * t2)
    pltpu.make_async_copy(gate_v, gate_hbm, gsem).start()

    def act_in(c, slot):
        return pltpu.make_async_copy(
            act_hbm.at[pl.ds(c * _CB, _CB)], act_b.at[slot], in_sems.at[slot])

    _H = _CB // 4

    def disp_out_h(c, slot, h, sems):
        return pltpu.make_async_copy(
            disp_b.at[slot, pl.ds(h * _H, _H)],
            disp_hbm.at[pl.ds(c * _CB + h * _H, _H)], sems.at[slot])

    for c in range(_LOOK):
        act_in(c, c % _NBUF).start()

    def step(c, carry):
        slot = lax.rem(c, _NBUF)

        @pl.when(c + _LOOK < _NCH)
        def _():
            act_in(c + _LOOK, lax.rem(c + _LOOK, _NBUF)).start()

        act_in(c, slot).wait()

        @pl.when(c >= _NBUF)
        def _():
            for h in range(4):
                disp_out_h(c - _NBUF, slot, h, out_sems if h % 2 == 0 else out_sems2).wait()

        for h, sems in ((0, out_sems), (1, out_sems2), (2, out_sems), (3, out_sems2)):
            gate = gate_v[pl.ds(c * _CB + h * _H, _H), :]
            gm = jnp.where(gate >= 0.5, gate, 0.0)
            a = act_b[slot, pl.ds(h * _H, _H)]
            disp_b[slot, pl.ds(h * _H, _H)] = gm[:, :, None] * a[:, None, :]
            disp_out_h(c, slot, h, sems).start()
        return carry

    lax.fori_loop(0, _NCH, step, 0, unroll=False)

    for k in range(_NCH - _NBUF, _NCH):
        for h in range(4):
            disp_out_h(k, k % _NBUF, h, out_sems if h % 2 == 0 else out_sems2).wait()
    pltpu.make_async_copy(gate_v, gate_hbm, gsem).wait()


def kernel(x, act, batch_inds):
    gate, disp = pl.pallas_call(
        _body,
        in_specs=[
            pl.BlockSpec(memory_space=pl.ANY),
            pl.BlockSpec(memory_space=pl.ANY),
        ],
        out_specs=[
            pl.BlockSpec(memory_space=pl.ANY),
            pl.BlockSpec(memory_space=pl.ANY),
        ],
        out_shape=[
            jax.ShapeDtypeStruct((_N, _E), jnp.float32),
            jax.ShapeDtypeStruct((_N, _E, _D), jnp.float32),
        ],
        scratch_shapes=[
            pltpu.VMEM((_N, _E), jnp.float32),
            pltpu.VMEM((_N, _E), jnp.float32),
            pltpu.VMEM((_NBUF, _CB, _D), jnp.float32),
            pltpu.VMEM((_NBUF, _CB, _E, _D), jnp.float32),
            pltpu.SemaphoreType.DMA((_NBUF,)),
            pltpu.SemaphoreType.DMA((_NBUF,)),
            pltpu.SemaphoreType.DMA((_NBUF,)),
            pltpu.SemaphoreType.DMA,
        ],
    )(x, act)
    return gate, disp


# speedup vs baseline: 1.3895x; 1.0164x over previous
import jax
import jax.numpy as jnp
from jax import lax
from jax.experimental import pallas as pl
from jax.experimental.pallas import tpu as pltpu

_N, _E, _D = 4096, 8, 768
_CB = 512
_NCH = _N // _CB
_NBUF = 4
_LOOK = 3


def _body(x_hbm, act_hbm, gate_hbm, disp_hbm,
          x_v, gate_v, act_b, disp_b, in_sems, out_sems, out_sems2, gsem):
    pltpu.make_async_copy(x_hbm, x_v, gsem).start()
    pltpu.make_async_copy(x_hbm, x_v, gsem).wait()
    t = x_v[...] * 2.0
    t2 = t * t
    gate_v[...] = 1.0 / (1.0 + t2 * t2)
    pltpu.make_async_copy(gate_v, gate_hbm, gsem).start()

    def act_in(c, slot):
        return pltpu.make_async_copy(
            act_hbm.at[pl.ds(c * _CB, _CB)], act_b.at[slot], in_sems.at[slot])

    _H = _CB // 2

    def disp_out_h(c, slot, h, sems):
        return pltpu.make_async_copy(
            disp_b.at[slot, pl.ds(h * _H, _H)],
            disp_hbm.at[pl.ds(c * _CB + h * _H, _H)], sems.at[slot])

    for c in range(_LOOK):
        act_in(c, c % _NBUF).start()

    def step(c, carry):
        slot = lax.rem(c, _NBUF)

        @pl.when(c + _LOOK < _NCH)
        def _():
            act_in(c + _LOOK, lax.rem(c + _LOOK, _NBUF)).start()

        act_in(c, slot).wait()

        @pl.when(c >= _NBUF)
        def _():
            disp_out_h(c - _NBUF, slot, 0, out_sems).wait()
            disp_out_h(c - _NBUF, slot, 1, out_sems2).wait()

        for h, sems in ((0, out_sems), (1, out_sems2)):
            gate = gate_v[pl.ds(c * _CB + h * _H, _H), :]
            gm = jnp.where(gate >= 0.5, gate, 0.0)
            a = act_b[slot, pl.ds(h * _H, _H)]
            disp_b[slot, pl.ds(h * _H, _H)] = gm[:, :, None] * a[:, None, :]
            disp_out_h(c, slot, h, sems).start()
        return carry

    lax.fori_loop(0, _NCH, step, 0, unroll=False)

    for k in range(_NCH - _NBUF, _NCH):
        disp_out_h(k, k % _NBUF, 0, out_sems).wait()
        disp_out_h(k, k % _NBUF, 1, out_sems2).wait()
    pltpu.make_async_copy(gate_v, gate_hbm, gsem).wait()


def kernel(x, act, batch_inds):
    gate, disp = pl.pallas_call(
        _body,
        in_specs=[
            pl.BlockSpec(memory_space=pl.ANY),
            pl.BlockSpec(memory_space=pl.ANY),
        ],
        out_specs=[
            pl.BlockSpec(memory_space=pl.ANY),
            pl.BlockSpec(memory_space=pl.ANY),
        ],
        out_shape=[
            jax.ShapeDtypeStruct((_N, _E), jnp.float32),
            jax.ShapeDtypeStruct((_N, _E, _D), jnp.float32),
        ],
        scratch_shapes=[
            pltpu.VMEM((_N, _E), jnp.float32),
            pltpu.VMEM((_N, _E), jnp.float32),
            pltpu.VMEM((_NBUF, _CB, _D), jnp.float32),
            pltpu.VMEM((_NBUF, _CB, _E, _D), jnp.float32),
            pltpu.SemaphoreType.DMA((_NBUF,)),
            pltpu.SemaphoreType.DMA((_NBUF,)),
            pltpu.SemaphoreType.DMA((_NBUF,)),
            pltpu.SemaphoreType.DMA,
        ],
    )(x, act)
    return gate, disp


# speedup vs baseline: 1.4091x; 1.0142x over previous
import jax
import jax.numpy as jnp
from jax import lax
from jax.experimental import pallas as pl
from jax.experimental.pallas import tpu as pltpu

_N, _E, _D = 4096, 8, 768
_CB = 512
_NCH = _N // _CB
_NBUF = 3
_LOOK = 3


def _body(x_hbm, act_hbm, gate_hbm, disp_hbm,
          x_v, gate_v, act_b, disp_b, in_sems, out_sems, out_sems2, gsem):
    pltpu.make_async_copy(x_hbm, x_v, gsem).start()
    pltpu.make_async_copy(x_hbm, x_v, gsem).wait()
    t = x_v[...] * 2.0
    t2 = t * t
    gate_v[...] = 1.0 / (1.0 + t2 * t2)
    pltpu.make_async_copy(gate_v, gate_hbm, gsem).start()

    def act_in(c):
        return pltpu.make_async_copy(
            act_hbm.at[pl.ds(c * _CB, _CB)],
            act_b.at[pl.ds(c * _CB, _CB)], in_sems.at[c])

    _H = _CB // 2

    def disp_out_h(c, slot, h, sems):
        return pltpu.make_async_copy(
            disp_b.at[slot, pl.ds(h * _H, _H)],
            disp_hbm.at[pl.ds(c * _CB + h * _H, _H)], sems.at[slot])

    for c in range(_NCH):
        act_in(c).start()

    def step(c, carry):
        slot = lax.rem(c, _NBUF)

        act_in(c).wait()

        @pl.when(c >= _NBUF)
        def _():
            disp_out_h(c - _NBUF, slot, 0, out_sems).wait()
            disp_out_h(c - _NBUF, slot, 1, out_sems2).wait()

        for h, sems in ((0, out_sems), (1, out_sems2)):
            gate = gate_v[pl.ds(c * _CB + h * _H, _H), :]
            gm = jnp.where(gate >= 0.5, gate, 0.0)
            a = act_b[pl.ds(c * _CB + h * _H, _H)]
            disp_b[slot, pl.ds(h * _H, _H)] = gm[:, :, None] * a[:, None, :]
            disp_out_h(c, slot, h, sems).start()
        return carry

    lax.fori_loop(0, _NCH, step, 0, unroll=False)

    for k in range(_NCH - _NBUF, _NCH):
        disp_out_h(k, k % _NBUF, 0, out_sems).wait()
        disp_out_h(k, k % _NBUF, 1, out_sems2).wait()
    pltpu.make_async_copy(gate_v, gate_hbm, gsem).wait()


def kernel(x, act, batch_inds):
    gate, disp = pl.pallas_call(
        _body,
        in_specs=[
            pl.BlockSpec(memory_space=pl.ANY),
            pl.BlockSpec(memory_space=pl.ANY),
        ],
        out_specs=[
            pl.BlockSpec(memory_space=pl.ANY),
            pl.BlockSpec(memory_space=pl.ANY),
        ],
        out_shape=[
            jax.ShapeDtypeStruct((_N, _E), jnp.float32),
            jax.ShapeDtypeStruct((_N, _E, _D), jnp.float32),
        ],
        scratch_shapes=[
            pltpu.VMEM((_N, _E), jnp.float32),
            pltpu.VMEM((_N, _E), jnp.float32),
            pltpu.VMEM((_N, _D), jnp.float32),
            pltpu.VMEM((_NBUF, _CB, _E, _D), jnp.float32),
            pltpu.SemaphoreType.DMA((_NCH,)),
            pltpu.SemaphoreType.DMA((_NBUF,)),
            pltpu.SemaphoreType.DMA((_NBUF,)),
            pltpu.SemaphoreType.DMA,
        ],
    )(x, act)
    return gate, disp


# speedup vs baseline: 1.4099x; 1.0005x over previous
import jax
import jax.numpy as jnp
from jax import lax
from jax.experimental import pallas as pl
from jax.experimental.pallas import tpu as pltpu

_N, _E, _D = 4096, 8, 768
_CB = 256
_NCH = _N // _CB
_NBUF = 6
_LOOK = 3


def _body(x_hbm, act_hbm, gate_hbm, disp_hbm,
          x_v, gate_v, act_b, disp_b, in_sems, out_sems, out_sems2, gsem):
    pltpu.make_async_copy(x_hbm, x_v, gsem).start()
    pltpu.make_async_copy(x_hbm, x_v, gsem).wait()
    t = x_v[...] * 2.0
    t2 = t * t
    gate_v[...] = 1.0 / (1.0 + t2 * t2)
    pltpu.make_async_copy(gate_v, gate_hbm, gsem).start()

    def act_in(c):
        return pltpu.make_async_copy(
            act_hbm.at[pl.ds(c * _CB, _CB)],
            act_b.at[pl.ds(c * _CB, _CB)], in_sems.at[c])

    _H = _CB // 2

    def disp_out_h(c, slot, h, sems):
        return pltpu.make_async_copy(
            disp_b.at[slot, pl.ds(h * _H, _H)],
            disp_hbm.at[pl.ds(c * _CB + h * _H, _H)], sems.at[slot])

    for c in range(_NCH):
        act_in(c).start()

    def step(c, carry):
        slot = lax.rem(c, _NBUF)

        act_in(c).wait()

        @pl.when(c >= _NBUF)
        def _():
            disp_out_h(c - _NBUF, slot, 0, out_sems).wait()
            disp_out_h(c - _NBUF, slot, 1, out_sems2).wait()

        for h, sems in ((0, out_sems), (1, out_sems2)):
            gate = gate_v[pl.ds(c * _CB + h * _H, _H), :]
            gm = jnp.where(gate >= 0.5, gate, 0.0)
            a = act_b[pl.ds(c * _CB + h * _H, _H)]
            disp_b[slot, pl.ds(h * _H, _H)] = gm[:, :, None] * a[:, None, :]
            disp_out_h(c, slot, h, sems).start()
        return carry

    lax.fori_loop(0, _NCH, step, 0, unroll=False)

    for k in range(_NCH - _NBUF, _NCH):
        disp_out_h(k, k % _NBUF, 0, out_sems).wait()
        disp_out_h(k, k % _NBUF, 1, out_sems2).wait()
    pltpu.make_async_copy(gate_v, gate_hbm, gsem).wait()


def kernel(x, act, batch_inds):
    gate, disp = pl.pallas_call(
        _body,
        in_specs=[
            pl.BlockSpec(memory_space=pl.ANY),
            pl.BlockSpec(memory_space=pl.ANY),
        ],
        out_specs=[
            pl.BlockSpec(memory_space=pl.ANY),
            pl.BlockSpec(memory_space=pl.ANY),
        ],
        out_shape=[
            jax.ShapeDtypeStruct((_N, _E), jnp.float32),
            jax.ShapeDtypeStruct((_N, _E, _D), jnp.float32),
        ],
        scratch_shapes=[
            pltpu.VMEM((_N, _E), jnp.float32),
            pltpu.VMEM((_N, _E), jnp.float32),
            pltpu.VMEM((_N, _D), jnp.float32),
            pltpu.VMEM((_NBUF, _CB, _E, _D), jnp.float32),
            pltpu.SemaphoreType.DMA((_NCH,)),
            pltpu.SemaphoreType.DMA((_NBUF,)),
            pltpu.SemaphoreType.DMA((_NBUF,)),
            pltpu.SemaphoreType.DMA,
        ],
    )(x, act)
    return gate, disp
